# baseline (speedup 1.0000x reference)
import functools

import jax
import jax.numpy as jnp
from jax.experimental import pallas as pl

N = 4096
D_IN = 512
D_OUT = 512
K = 16
BLK_R = 128


def _embed_body(x_ref, w_ref, xt_ref, x2_ref):
    xt = jnp.dot(x_ref[...], w_ref[...], preferred_element_type=jnp.float32)
    xt_ref[...] = xt
    x2_ref[...] = jnp.sum(xt * xt, axis=1, keepdims=True)


def _knn_body(c_ref, a_ref, b_ref, x2a_ref, x2b_ref, q_ref, idx_ref, lp_ref):
    c = c_ref[...]
    a = a_ref[...]
    g = jax.lax.dot_general(
        a, b_ref[...], (((1,), (1,)), ((), ())),
        preferred_element_type=jnp.float32)
    d = jnp.maximum(x2a_ref[...] + x2b_ref[...] - 2.0 * g, 0.0)
    q = q_ref[0]
    gumbel = -jnp.log(-jnp.log(q + 1e-8))
    p = d * c - gumbel
    inf = jnp.float32(jnp.inf)
    iota = jax.lax.broadcasted_iota(jnp.int32, p.shape, 1).astype(jnp.float32)
    idx_cols = []
    lp_cols = []
    for _ in range(K):
        m = jnp.min(p, axis=1, keepdims=True)
        eq = p == m
        idx = jnp.min(jnp.where(eq, iota, inf), axis=1, keepdims=True)
        sel = iota == idx
        gsel = jnp.min(jnp.where(sel, gumbel, inf), axis=1, keepdims=True)
        idx_cols.append(idx)
        lp_cols.append(-(m + gsel))
        p = jnp.where(sel, inf, p)
    idx_ref[...] = jnp.concatenate(idx_cols, axis=1).astype(jnp.int32)
    lp_ref[...] = jnp.concatenate(lp_cols, axis=1)


@functools.partial(jax.jit, static_argnames=())
def kernel(X, W, q, t):
    nb = N // BLK_R
    xt, x2 = pl.pallas_call(
        _embed_body,
        grid=(nb,),
        in_specs=[
            pl.BlockSpec((BLK_R, D_IN), lambda i: (i, 0)),
            pl.BlockSpec((D_IN, D_OUT), lambda i: (0, 0)),
        ],
        out_specs=[
            pl.BlockSpec((BLK_R, D_OUT), lambda i: (i, 0)),
            pl.BlockSpec((BLK_R, 1), lambda i: (i, 0)),
        ],
        out_shape=[
            jax.ShapeDtypeStruct((N, D_OUT), jnp.float32),
            jax.ShapeDtypeStruct((N, 1), jnp.float32),
        ],
    )(X, W)

    c = jnp.exp(jnp.clip(t, -5.0, 5.0)).reshape(1, 1).astype(jnp.float32)
    x2_row = x2.reshape(1, N)

    idx, lp = pl.pallas_call(
        _knn_body,
        grid=(nb,),
        in_specs=[
            pl.BlockSpec((1, 1), lambda i: (0, 0)),
            pl.BlockSpec((BLK_R, D_OUT), lambda i: (i, 0)),
            pl.BlockSpec((N, D_OUT), lambda i: (0, 0)),
            pl.BlockSpec((BLK_R, 1), lambda i: (i, 0)),
            pl.BlockSpec((1, N), lambda i: (0, 0)),
            pl.BlockSpec((1, BLK_R, N), lambda i: (0, i, 0)),
        ],
        out_specs=[
            pl.BlockSpec((BLK_R, K), lambda i: (i, 0)),
            pl.BlockSpec((BLK_R, K), lambda i: (i, 0)),
        ],
        out_shape=[
            jax.ShapeDtypeStruct((N, K), jnp.int32),
            jax.ShapeDtypeStruct((N, K), jnp.float32),
        ],
    )(c, xt, xt, x2, x2_row, q)

    rows = jnp.broadcast_to(
        jnp.arange(N, dtype=jnp.int32).reshape(N, 1), (N, K))
    edge_index = jnp.stack([idx.reshape(-1), rows.reshape(-1)], axis=0)
    return xt[None], edge_index, lp[None]


# speedup vs baseline: 1.0776x; 1.0776x over previous
import functools

import jax
import jax.numpy as jnp
from jax.experimental import pallas as pl

N = 4096
D_IN = 512
D_OUT = 512
K = 16
BLK_R = 256


def _embed_body(x_ref, w_ref, xt_ref, x2_ref):
    xt = jnp.dot(x_ref[...], w_ref[...], preferred_element_type=jnp.float32)
    xt_ref[...] = xt
    x2_ref[...] = jnp.sum(xt * xt, axis=1, keepdims=True)


def _knn_body(c_ref, a_ref, b_ref, x2a_ref, x2b_ref, q_ref, idx_ref, lp_ref):
    c = c_ref[...]
    a = a_ref[...]
    g = jax.lax.dot_general(
        a, b_ref[...], (((1,), (1,)), ((), ())),
        preferred_element_type=jnp.float32)
    d = jnp.maximum(x2a_ref[...] + x2b_ref[...] - 2.0 * g, 0.0)
    q = q_ref[0]
    gumbel = -jnp.log(-jnp.log(q + 1e-8))
    p = d * c - gumbel
    inf = jnp.float32(jnp.inf)
    iota = jax.lax.broadcasted_iota(jnp.int32, p.shape, 1).astype(jnp.float32)
    idx_cols = []
    lp_cols = []
    for _ in range(K):
        m = jnp.min(p, axis=1, keepdims=True)
        eq = p == m
        idx = jnp.min(jnp.where(eq, iota, inf), axis=1, keepdims=True)
        sel = iota == idx
        gsel = jnp.min(jnp.where(sel, gumbel, inf), axis=1, keepdims=True)
        idx_cols.append(idx)
        lp_cols.append(-(m + gsel))
        p = jnp.where(sel, inf, p)
    idx_ref[...] = jnp.concatenate(idx_cols, axis=1).astype(jnp.int32)
    lp_ref[...] = jnp.concatenate(lp_cols, axis=1)


@functools.partial(jax.jit, static_argnames=())
def kernel(X, W, q, t):
    nb = N // BLK_R
    xt, x2 = pl.pallas_call(
        _embed_body,
        grid=(nb,),
        in_specs=[
            pl.BlockSpec((BLK_R, D_IN), lambda i: (i, 0)),
            pl.BlockSpec((D_IN, D_OUT), lambda i: (0, 0)),
        ],
        out_specs=[
            pl.BlockSpec((BLK_R, D_OUT), lambda i: (i, 0)),
            pl.BlockSpec((BLK_R, 1), lambda i: (i, 0)),
        ],
        out_shape=[
            jax.ShapeDtypeStruct((N, D_OUT), jnp.float32),
            jax.ShapeDtypeStruct((N, 1), jnp.float32),
        ],
    )(X, W)

    c = jnp.exp(jnp.clip(t, -5.0, 5.0)).reshape(1, 1).astype(jnp.float32)
    x2_row = x2.reshape(1, N)

    idx, lp = pl.pallas_call(
        _knn_body,
        grid=(nb,),
        in_specs=[
            pl.BlockSpec((1, 1), lambda i: (0, 0)),
            pl.BlockSpec((BLK_R, D_OUT), lambda i: (i, 0)),
            pl.BlockSpec((N, D_OUT), lambda i: (0, 0)),
            pl.BlockSpec((BLK_R, 1), lambda i: (i, 0)),
            pl.BlockSpec((1, N), lambda i: (0, 0)),
            pl.BlockSpec((1, BLK_R, N), lambda i: (0, i, 0)),
        ],
        out_specs=[
            pl.BlockSpec((BLK_R, K), lambda i: (i, 0)),
            pl.BlockSpec((BLK_R, K), lambda i: (i, 0)),
        ],
        out_shape=[
            jax.ShapeDtypeStruct((N, K), jnp.int32),
            jax.ShapeDtypeStruct((N, K), jnp.float32),
        ],
    )(c, xt, xt, x2, x2_row, q)

    rows = jnp.broadcast_to(
        jnp.arange(N, dtype=jnp.int32).reshape(N, 1), (N, K))
    edge_index = jnp.stack([idx.reshape(-1), rows.reshape(-1)], axis=0)
    return xt[None], edge_index, lp[None]
